# baseline (speedup 1.0000x reference)
import functools

import jax
import jax.numpy as jnp
import numpy as np
from jax import lax
from jax.experimental import pallas as pl
from jax.experimental.pallas import tpu as pltpu
from jax.experimental.pallas import tpu_sc as plsc

S, B, H = 2048, 1, 1024
NH, NKV, DH = 16, 16, 64
E, TOPK, FFN = 16, 2, 1024

BLK = 128
PT = TOPK * S + E * BLK
NB = PT // BLK
NC, NS = 2, 16
NW = NC * NS
TPW = S // NW

_HIGH = jax.lax.Precision.DEFAULT


def _dot(a, b, dims, precision=_HIGH):
    return jax.lax.dot_general(a, b, (dims, ((), ())), precision=precision,
                               preferred_element_type=jnp.float32)


_TB1 = 256


def _ln_qkv_body(x_ref, w_ref, b_ref, qkvw_ref, o_ref):
    x = x_ref[...]
    mu = jnp.mean(x, axis=-1, keepdims=True)
    var = jnp.mean((x - mu) ** 2, axis=-1, keepdims=True)
    xn = (x - mu) / jnp.sqrt(var + 1e-5) * w_ref[...] + b_ref[...]
    o_ref[...] = _dot(xn, qkvw_ref[...], ((1,), (1,)))


_QB = 256


def _attn_body(q_ref, k_ref, v_ref, o_ref, *, scale):
    qb = q_ref[:, 0, 0, :]
    i = pl.program_id(1)

    def compute(kl):
        kb = k_ref[:kl, 0, 0, :]
        vb = v_ref[:kl, 0, 0, :]
        s = _dot(qb, kb, ((1,), (1,))) * scale
        row = i * _QB + jax.lax.broadcasted_iota(jnp.int32, (_QB, kl), 0)
        col = jax.lax.broadcasted_iota(jnp.int32, (_QB, kl), 1)
        s = jnp.where(col <= row, s, jnp.float32(-1e9))
        m = jnp.max(s, axis=-1, keepdims=True)
        p = jnp.exp(s - m)
        den = jnp.sum(p, axis=-1, keepdims=True)
        o_ref[:, 0, 0, :] = _dot(p, vb, ((1,), (0,))) / den

    for b in range(S // _QB):
        @pl.when(i == b)
        def _(b=b):
            compute((b + 1) * _QB)


_TB3 = 256


def _proj_gates_body(attn_ref, hid_ref, projw_ref, ln2w_ref, ln2b_ref,
                     routw_ref, h2_ref, ln2o_ref, oh1_ref, oh2_ref,
                     g1_ref, g2_ref):
    proj = _dot(attn_ref[...], projw_ref[...], ((1,), (1,)))
    h2 = hid_ref[...] + proj
    h2_ref[...] = h2
    mu = jnp.mean(h2, axis=-1, keepdims=True)
    var = jnp.mean((h2 - mu) ** 2, axis=-1, keepdims=True)
    ln2 = (h2 - mu) / jnp.sqrt(var + 1e-5) * ln2w_ref[...] + ln2b_ref[...]
    ln2o_ref[...] = ln2
    logits = _dot(ln2, routw_ref[...], ((1,), (1,)))
    m = jnp.max(logits, axis=-1, keepdims=True)
    ex = jnp.exp(logits - m)
    p = ex / jnp.sum(ex, axis=-1, keepdims=True)
    lanes = jax.lax.broadcasted_iota(jnp.int32, (_TB3, E), 1)
    i1 = jnp.argmax(p, axis=-1)[:, None]
    m1 = lanes == i1
    p_wo1 = jnp.where(m1, jnp.float32(-1.0), p)
    i2 = jnp.argmax(p_wo1, axis=-1)[:, None]
    m2 = lanes == i2
    oh1_ref[...] = m1.astype(jnp.float32)
    oh2_ref[...] = m2.astype(jnp.float32)
    g1 = jnp.sum(jnp.where(m1, p, 0.0), axis=-1, keepdims=True)
    g2 = jnp.sum(jnp.where(m2, p, 0.0), axis=-1, keepdims=True)
    g1_ref[...] = jnp.broadcast_to(g1, (_TB3, 16))
    g2_ref[...] = jnp.broadcast_to(g2, (_TB3, 16))


def _shift_down(x, d):
    return jnp.concatenate([jnp.zeros((d, E), jnp.float32), x[:S - d]], axis=0)


def _route_meta_body(oh1_ref, oh2_ref, pos0_ref, pos1_ref, start_ref):
    oh1 = oh1_ref[...]
    oh2 = oh2_ref[...]

    def incl_scan(x):
        acc = x
        d = 1
        while d < S:
            acc = acc + _shift_down(acc, d)
            d *= 2
        return acc

    inc1 = incl_scan(oh1)
    inc2 = incl_scan(oh2)
    within1 = inc1 - oh1
    within2 = inc2 - oh2
    counts0 = inc1[S - 1:S, :]
    counts = counts0 + inc2[S - 1:S, :]
    ci = counts.astype(jnp.int32)
    pc = ((ci + (BLK - 1)) // BLK) * BLK
    ltri = (jax.lax.broadcasted_iota(jnp.int32, (E, E), 0)
            < jax.lax.broadcasted_iota(jnp.int32, (E, E), 1)).astype(jnp.float32)
    start_f = _dot(pc.astype(jnp.float32), ltri, ((1,), (0,)))
    start_ref[...] = start_f.astype(jnp.int32)
    pos0 = jnp.sum(oh1 * (start_f + within1), axis=1, keepdims=True)
    pos1 = jnp.sum(oh2 * (start_f + counts0 + within2), axis=1, keepdims=True)
    pos0_ref[...] = pos0.astype(jnp.int32)
    pos1_ref[...] = pos1.astype(jnp.int32)


def _gmm_body(start_ref, x_ref, w1_ref, w2_ref, y_ref):
    h = _dot(x_ref[...], w1_ref[0], ((1,), (0,)))
    h = h * jax.nn.sigmoid(h)
    y_ref[...] = _dot(h, w2_ref[0], ((1,), (0,)))


def _expert_of(b, start_ref):
    acc = jnp.int32(0)
    for e in range(E):
        acc = acc + (start_ref[e] <= b * BLK).astype(jnp.int32)
    return acc - 1


def _sc_mesh():
    return plsc.VectorSubcoreMesh(core_axis_name="c", subcore_axis_name="s")


def _dispatch_body(ln2_hbm, pos0_hbm, pos1_hbm, xs_hbm,
                   idx0_v, idx1_v, rows_v, sem0, sem1):
    wid = lax.axis_index("s") * NC + lax.axis_index("c")
    base = wid * TPW
    pltpu.sync_copy(pos0_hbm.at[pl.ds(base, TPW)], idx0_v)
    pltpu.sync_copy(pos1_hbm.at[pl.ds(base, TPW)], idx1_v)
    pltpu.sync_copy(ln2_hbm.at[pl.ds(base, TPW)], rows_v)
    c0 = pltpu.async_copy(rows_v, xs_hbm.at[idx0_v], sem0)
    c1 = pltpu.async_copy(rows_v, xs_hbm.at[idx1_v], sem1)
    c0.wait()
    c1.wait()


_CHUNK = 32


def _combine_body(y_hbm, h2_hbm, pos0_hbm, pos1_hbm, g1_hbm, g2_hbm, out_hbm,
                  i0_v, i1_v, y0_v, y1_v, h2_v, g1_v, g2_v, sem0, sem1):
    wid = lax.axis_index("s") * NC + lax.axis_index("c")
    for c in range(TPW // _CHUNK):
        base = wid * TPW + c * _CHUNK
        pltpu.sync_copy(pos0_hbm.at[pl.ds(base, _CHUNK)], i0_v)
        pltpu.sync_copy(pos1_hbm.at[pl.ds(base, _CHUNK)], i1_v)
        pltpu.sync_copy(g1_hbm.at[pl.ds(base, _CHUNK)], g1_v)
        pltpu.sync_copy(g2_hbm.at[pl.ds(base, _CHUNK)], g2_v)
        c0 = pltpu.async_copy(y_hbm.at[i0_v], y0_v, sem0)
        c1 = pltpu.async_copy(y_hbm.at[i1_v], y1_v, sem1)
        pltpu.sync_copy(h2_hbm.at[pl.ds(base, _CHUNK)], h2_v)
        c0.wait()
        c1.wait()

        def body(i, carry):
            g1vec = g1_v[i, :]
            g2vec = g2_v[i, :]
            for j in range(H // 16):
                sl = pl.ds(j * 16, 16)
                h2_v[i, sl] = (h2_v[i, sl] + g1vec * y0_v[i, sl]
                               + g2vec * y1_v[i, sl])
            return carry

        lax.fori_loop(0, _CHUNK, body, 0)
        pltpu.sync_copy(h2_v, out_hbm.at[pl.ds(base, _CHUNK)])


def kernel(hidden_states, ln1_weight, ln1_bias, ln2_weight, ln2_bias,
           qkv_weight, proj_weight, router_weight, moe_w1, moe_w2):
    scale = 1.0 / np.sqrt(DH)
    x = hidden_states.reshape(S, H)

    qkv = pl.pallas_call(
        _ln_qkv_body,
        grid=(S // _TB1,),
        in_specs=[
            pl.BlockSpec((_TB1, H), lambda i: (i, 0)),
            pl.BlockSpec((1, H), lambda i: (0, 0)),
            pl.BlockSpec((1, H), lambda i: (0, 0)),
            pl.BlockSpec(((NH + 2 * NKV) * DH, H), lambda i: (0, 0)),
        ],
        out_specs=pl.BlockSpec((_TB1, (NH + 2 * NKV) * DH), lambda i: (i, 0)),
        out_shape=jax.ShapeDtypeStruct((S, (NH + 2 * NKV) * DH), jnp.float32),
    )(x, ln1_weight.reshape(1, H), ln1_bias.reshape(1, H), qkv_weight)

    nqkv = NH + 2 * NKV
    qkv4 = qkv.reshape(S, nqkv, 1, DH)

    ctx4 = pl.pallas_call(
        functools.partial(_attn_body, scale=scale),
        grid=(NH, S // _QB),
        in_specs=[
            pl.BlockSpec((_QB, 1, 1, DH), lambda h, i: (i, h, 0, 0)),
            pl.BlockSpec((S, 1, 1, DH), lambda h, i: (0, NH + h, 0, 0)),
            pl.BlockSpec((S, 1, 1, DH), lambda h, i: (0, NH + NKV + h, 0, 0)),
        ],
        out_specs=pl.BlockSpec((_QB, 1, 1, DH), lambda h, i: (i, h, 0, 0)),
        out_shape=jax.ShapeDtypeStruct((S, NH, 1, DH), jnp.float32),
    )(qkv4, qkv4, qkv4)

    attn_flat = ctx4.reshape(S, NH * DH)

    h2, ln2_out, oh1, oh2, g1c, g2c = pl.pallas_call(
        _proj_gates_body,
        grid=(S // _TB3,),
        in_specs=[
            pl.BlockSpec((_TB3, NH * DH), lambda i: (i, 0)),
            pl.BlockSpec((_TB3, H), lambda i: (i, 0)),
            pl.BlockSpec((H, NH * DH), lambda i: (0, 0)),
            pl.BlockSpec((1, H), lambda i: (0, 0)),
            pl.BlockSpec((1, H), lambda i: (0, 0)),
            pl.BlockSpec((E, H), lambda i: (0, 0)),
        ],
        out_specs=[
            pl.BlockSpec((_TB3, H), lambda i: (i, 0)),
            pl.BlockSpec((_TB3, H), lambda i: (i, 0)),
            pl.BlockSpec((_TB3, E), lambda i: (i, 0)),
            pl.BlockSpec((_TB3, E), lambda i: (i, 0)),
            pl.BlockSpec((_TB3, 16), lambda i: (i, 0)),
            pl.BlockSpec((_TB3, 16), lambda i: (i, 0)),
        ],
        out_shape=[
            jax.ShapeDtypeStruct((S, H), jnp.float32),
            jax.ShapeDtypeStruct((S, H), jnp.float32),
            jax.ShapeDtypeStruct((S, E), jnp.float32),
            jax.ShapeDtypeStruct((S, E), jnp.float32),
            jax.ShapeDtypeStruct((S, 16), jnp.float32),
            jax.ShapeDtypeStruct((S, 16), jnp.float32),
        ],
    )(attn_flat, x, proj_weight, ln2_weight.reshape(1, H),
      ln2_bias.reshape(1, H), router_weight)

    pos0c, pos1c, start_row = pl.pallas_call(
        _route_meta_body,
        grid=(1,),
        in_specs=[
            pl.BlockSpec((S, E), lambda i: (0, 0)),
            pl.BlockSpec((S, E), lambda i: (0, 0)),
        ],
        out_specs=[
            pl.BlockSpec((S, 1), lambda i: (0, 0)),
            pl.BlockSpec((S, 1), lambda i: (0, 0)),
            pl.BlockSpec((1, E), lambda i: (0, 0)),
        ],
        out_shape=[
            jax.ShapeDtypeStruct((S, 1), jnp.int32),
            jax.ShapeDtypeStruct((S, 1), jnp.int32),
            jax.ShapeDtypeStruct((1, E), jnp.int32),
        ],
    )(oh1, oh2)

    pos0 = pos0c.reshape(S)
    pos1 = pos1c.reshape(S)
    start_pad = start_row.reshape(E)

    dispatch = functools.partial(
        pl.kernel,
        mesh=_sc_mesh(),
        out_type=jax.ShapeDtypeStruct((PT, H), jnp.float32),
        scratch_types=[
            pltpu.VMEM((TPW,), jnp.int32),
            pltpu.VMEM((TPW,), jnp.int32),
            pltpu.VMEM((TPW, H), jnp.float32),
            pltpu.SemaphoreType.DMA,
            pltpu.SemaphoreType.DMA,
        ],
    )(_dispatch_body)
    xs = dispatch(ln2_out, pos0, pos1)

    y = pl.pallas_call(
        _gmm_body,
        grid_spec=pltpu.PrefetchScalarGridSpec(
            num_scalar_prefetch=1,
            grid=(NB,),
            in_specs=[
                pl.BlockSpec((BLK, H), lambda b, s: (b, 0)),
                pl.BlockSpec((1, H, FFN), lambda b, s: (_expert_of(b, s), 0, 0)),
                pl.BlockSpec((1, FFN, H), lambda b, s: (_expert_of(b, s), 0, 0)),
            ],
            out_specs=pl.BlockSpec((BLK, H), lambda b, s: (b, 0)),
        ),
        out_shape=jax.ShapeDtypeStruct((PT, H), jnp.float32),
    )(start_pad, xs, moe_w1, moe_w2)

    combine = functools.partial(
        pl.kernel,
        mesh=_sc_mesh(),
        out_type=jax.ShapeDtypeStruct((S, H), jnp.float32),
        scratch_types=[
            pltpu.VMEM((_CHUNK,), jnp.int32),
            pltpu.VMEM((_CHUNK,), jnp.int32),
            pltpu.VMEM((_CHUNK, H), jnp.float32),
            pltpu.VMEM((_CHUNK, H), jnp.float32),
            pltpu.VMEM((_CHUNK, H), jnp.float32),
            pltpu.VMEM((_CHUNK, 16), jnp.float32),
            pltpu.VMEM((_CHUNK, 16), jnp.float32),
            pltpu.SemaphoreType.DMA,
            pltpu.SemaphoreType.DMA,
        ],
    )(_combine_body)
    out = combine(y, h2, pos0, pos1, g1c, g2c)

    return out.reshape(S, B, H)


# speedup vs baseline: 1.2755x; 1.2755x over previous
import functools

import jax
import jax.numpy as jnp
import numpy as np
from jax import lax
from jax.experimental import pallas as pl
from jax.experimental.pallas import tpu as pltpu
from jax.experimental.pallas import tpu_sc as plsc

S, B, H = 2048, 1, 1024
NH, NKV, DH = 16, 16, 64
E, TOPK, FFN = 16, 2, 1024

BLK = 128
PT = TOPK * S + E * BLK
NB = PT // BLK
NC, NS = 2, 16
NW = NC * NS
TPW = S // NW

_HIGH = jax.lax.Precision.DEFAULT


def _dot(a, b, dims, precision=_HIGH):
    return jax.lax.dot_general(a, b, (dims, ((), ())), precision=precision,
                               preferred_element_type=jnp.float32)


_TB1 = 256


def _ln_qkv_body(x_ref, w_ref, b_ref, qkvw_ref, o_ref):
    x = x_ref[...]
    mu = jnp.mean(x, axis=-1, keepdims=True)
    var = jnp.mean((x - mu) ** 2, axis=-1, keepdims=True)
    xn = (x - mu) / jnp.sqrt(var + 1e-5) * w_ref[...] + b_ref[...]
    o_ref[...] = _dot(xn, qkvw_ref[...], ((1,), (1,)))


_QB = 256


def _attn_body(q_ref, k_ref, v_ref, o_ref, *, scale):
    qb = q_ref[0]
    i = pl.program_id(1)

    def compute(kl):
        kb = k_ref[0, :kl, :]
        vb = v_ref[0, :kl, :]
        s = _dot(qb, kb, ((1,), (1,))) * scale
        row = i * _QB + jax.lax.broadcasted_iota(jnp.int32, (_QB, kl), 0)
        col = jax.lax.broadcasted_iota(jnp.int32, (_QB, kl), 1)
        s = jnp.where(col <= row, s, jnp.float32(-1e9))
        m = jnp.max(s, axis=-1, keepdims=True)
        p = jnp.exp(s - m)
        den = jnp.sum(p, axis=-1, keepdims=True)
        o_ref[0] = _dot(p, vb, ((1,), (0,))) / den

    for b in range(S // _QB):
        @pl.when(i == b)
        def _(b=b):
            compute((b + 1) * _QB)


_TB3 = 256


def _proj_gates_body(attn_ref, hid_ref, projw_ref, ln2w_ref, ln2b_ref,
                     routw_ref, h2_ref, ln2o_ref, oh1_ref, oh2_ref,
                     g1_ref, g2_ref):
    proj = _dot(attn_ref[...], projw_ref[...], ((1,), (1,)))
    h2 = hid_ref[...] + proj
    h2_ref[...] = h2
    mu = jnp.mean(h2, axis=-1, keepdims=True)
    var = jnp.mean((h2 - mu) ** 2, axis=-1, keepdims=True)
    ln2 = (h2 - mu) / jnp.sqrt(var + 1e-5) * ln2w_ref[...] + ln2b_ref[...]
    ln2o_ref[...] = ln2
    logits = _dot(ln2, routw_ref[...], ((1,), (1,)))
    m = jnp.max(logits, axis=-1, keepdims=True)
    ex = jnp.exp(logits - m)
    p = ex / jnp.sum(ex, axis=-1, keepdims=True)
    lanes = jax.lax.broadcasted_iota(jnp.int32, (_TB3, E), 1)
    i1 = jnp.argmax(p, axis=-1)[:, None]
    m1 = lanes == i1
    p_wo1 = jnp.where(m1, jnp.float32(-1.0), p)
    i2 = jnp.argmax(p_wo1, axis=-1)[:, None]
    m2 = lanes == i2
    oh1_ref[...] = m1.astype(jnp.float32)
    oh2_ref[...] = m2.astype(jnp.float32)
    g1 = jnp.sum(jnp.where(m1, p, 0.0), axis=-1, keepdims=True)
    g2 = jnp.sum(jnp.where(m2, p, 0.0), axis=-1, keepdims=True)
    g1_ref[...] = jnp.broadcast_to(g1, (_TB3, 16))
    g2_ref[...] = jnp.broadcast_to(g2, (_TB3, 16))


def _shift_down(x, d):
    return jnp.concatenate([jnp.zeros((d, E), jnp.float32), x[:S - d]], axis=0)


def _route_meta_body(oh1_ref, oh2_ref, pos0_ref, pos1_ref, start_ref):
    oh1 = oh1_ref[...]
    oh2 = oh2_ref[...]

    def incl_scan(x):
        acc = x
        d = 1
        while d < S:
            acc = acc + _shift_down(acc, d)
            d *= 2
        return acc

    inc1 = incl_scan(oh1)
    inc2 = incl_scan(oh2)
    within1 = inc1 - oh1
    within2 = inc2 - oh2
    counts0 = inc1[S - 1:S, :]
    counts = counts0 + inc2[S - 1:S, :]
    ci = counts.astype(jnp.int32)
    pc = ((ci + (BLK - 1)) // BLK) * BLK
    ltri = (jax.lax.broadcasted_iota(jnp.int32, (E, E), 0)
            < jax.lax.broadcasted_iota(jnp.int32, (E, E), 1)).astype(jnp.float32)
    start_f = _dot(pc.astype(jnp.float32), ltri, ((1,), (0,)))
    start_ref[...] = start_f.astype(jnp.int32)
    pos0 = jnp.sum(oh1 * (start_f + within1), axis=1, keepdims=True)
    pos1 = jnp.sum(oh2 * (start_f + counts0 + within2), axis=1, keepdims=True)
    pos0_ref[...] = pos0.astype(jnp.int32)
    pos1_ref[...] = pos1.astype(jnp.int32)


def _gmm_body(start_ref, x_ref, w1_ref, w2_ref, y_ref):
    h = _dot(x_ref[...], w1_ref[0], ((1,), (0,)))
    h = h * jax.nn.sigmoid(h)
    y_ref[...] = _dot(h, w2_ref[0], ((1,), (0,)))


def _expert_of(b, start_ref):
    acc = jnp.int32(0)
    for e in range(E):
        acc = acc + (start_ref[e] <= b * BLK).astype(jnp.int32)
    return acc - 1


def _sc_mesh():
    return plsc.VectorSubcoreMesh(core_axis_name="c", subcore_axis_name="s")


def _dispatch_body(ln2_hbm, pos0_hbm, pos1_hbm, xs_hbm,
                   idx0_v, idx1_v, rows_v, sem0, sem1):
    wid = lax.axis_index("s") * NC + lax.axis_index("c")
    base = wid * TPW
    pltpu.sync_copy(pos0_hbm.at[pl.ds(base, TPW)], idx0_v)
    pltpu.sync_copy(pos1_hbm.at[pl.ds(base, TPW)], idx1_v)
    pltpu.sync_copy(ln2_hbm.at[pl.ds(base, TPW)], rows_v)
    c0 = pltpu.async_copy(rows_v, xs_hbm.at[idx0_v], sem0)
    c1 = pltpu.async_copy(rows_v, xs_hbm.at[idx1_v], sem1)
    c0.wait()
    c1.wait()


_CHUNK = 32


def _combine_body(y_hbm, h2_hbm, pos0_hbm, pos1_hbm, g1_hbm, g2_hbm, out_hbm,
                  i0_v, i1_v, y0_v, y1_v, h2_v, g1_v, g2_v, sem0, sem1):
    wid = lax.axis_index("s") * NC + lax.axis_index("c")
    for c in range(TPW // _CHUNK):
        base = wid * TPW + c * _CHUNK
        pltpu.sync_copy(pos0_hbm.at[pl.ds(base, _CHUNK)], i0_v)
        pltpu.sync_copy(pos1_hbm.at[pl.ds(base, _CHUNK)], i1_v)
        pltpu.sync_copy(g1_hbm.at[pl.ds(base, _CHUNK)], g1_v)
        pltpu.sync_copy(g2_hbm.at[pl.ds(base, _CHUNK)], g2_v)
        c0 = pltpu.async_copy(y_hbm.at[i0_v], y0_v, sem0)
        c1 = pltpu.async_copy(y_hbm.at[i1_v], y1_v, sem1)
        pltpu.sync_copy(h2_hbm.at[pl.ds(base, _CHUNK)], h2_v)
        c0.wait()
        c1.wait()

        def body(i, carry):
            g1vec = g1_v[i, :]
            g2vec = g2_v[i, :]
            for j in range(H // 16):
                sl = pl.ds(j * 16, 16)
                h2_v[i, sl] = (h2_v[i, sl] + g1vec * y0_v[i, sl]
                               + g2vec * y1_v[i, sl])
            return carry

        lax.fori_loop(0, _CHUNK, body, 0)
        pltpu.sync_copy(h2_v, out_hbm.at[pl.ds(base, _CHUNK)])


def kernel(hidden_states, ln1_weight, ln1_bias, ln2_weight, ln2_bias,
           qkv_weight, proj_weight, router_weight, moe_w1, moe_w2):
    scale = 1.0 / np.sqrt(DH)
    x = hidden_states.reshape(S, H)

    qkv = pl.pallas_call(
        _ln_qkv_body,
        grid=(S // _TB1,),
        in_specs=[
            pl.BlockSpec((_TB1, H), lambda i: (i, 0)),
            pl.BlockSpec((1, H), lambda i: (0, 0)),
            pl.BlockSpec((1, H), lambda i: (0, 0)),
            pl.BlockSpec(((NH + 2 * NKV) * DH, H), lambda i: (0, 0)),
        ],
        out_specs=pl.BlockSpec((_TB1, (NH + 2 * NKV) * DH), lambda i: (i, 0)),
        out_shape=jax.ShapeDtypeStruct((S, (NH + 2 * NKV) * DH), jnp.float32),
    )(x, ln1_weight.reshape(1, H), ln1_bias.reshape(1, H), qkv_weight)

    q = qkv[:, :NH * DH].reshape(S, NH, DH).transpose(1, 0, 2)
    k = qkv[:, NH * DH:(NH + NKV) * DH].reshape(S, NKV, DH).transpose(1, 0, 2)
    v = qkv[:, (NH + NKV) * DH:].reshape(S, NKV, DH).transpose(1, 0, 2)

    ctx = pl.pallas_call(
        functools.partial(_attn_body, scale=scale),
        grid=(NH, S // _QB),
        in_specs=[
            pl.BlockSpec((1, _QB, DH), lambda h, i: (h, i, 0)),
            pl.BlockSpec((1, S, DH), lambda h, i: (h, 0, 0)),
            pl.BlockSpec((1, S, DH), lambda h, i: (h, 0, 0)),
        ],
        out_specs=pl.BlockSpec((1, _QB, DH), lambda h, i: (h, i, 0)),
        out_shape=jax.ShapeDtypeStruct((NH, S, DH), jnp.float32),
    )(q, k, v)

    attn_flat = ctx.transpose(1, 0, 2).reshape(S, NH * DH)

    h2, ln2_out, oh1, oh2, g1c, g2c = pl.pallas_call(
        _proj_gates_body,
        grid=(S // _TB3,),
        in_specs=[
            pl.BlockSpec((_TB3, NH * DH), lambda i: (i, 0)),
            pl.BlockSpec((_TB3, H), lambda i: (i, 0)),
            pl.BlockSpec((H, NH * DH), lambda i: (0, 0)),
            pl.BlockSpec((1, H), lambda i: (0, 0)),
            pl.BlockSpec((1, H), lambda i: (0, 0)),
            pl.BlockSpec((E, H), lambda i: (0, 0)),
        ],
        out_specs=[
            pl.BlockSpec((_TB3, H), lambda i: (i, 0)),
            pl.BlockSpec((_TB3, H), lambda i: (i, 0)),
            pl.BlockSpec((_TB3, E), lambda i: (i, 0)),
            pl.BlockSpec((_TB3, E), lambda i: (i, 0)),
            pl.BlockSpec((_TB3, 16), lambda i: (i, 0)),
            pl.BlockSpec((_TB3, 16), lambda i: (i, 0)),
        ],
        out_shape=[
            jax.ShapeDtypeStruct((S, H), jnp.float32),
            jax.ShapeDtypeStruct((S, H), jnp.float32),
            jax.ShapeDtypeStruct((S, E), jnp.float32),
            jax.ShapeDtypeStruct((S, E), jnp.float32),
            jax.ShapeDtypeStruct((S, 16), jnp.float32),
            jax.ShapeDtypeStruct((S, 16), jnp.float32),
        ],
    )(attn_flat, x, proj_weight, ln2_weight.reshape(1, H),
      ln2_bias.reshape(1, H), router_weight)

    pos0c, pos1c, start_row = pl.pallas_call(
        _route_meta_body,
        grid=(1,),
        in_specs=[
            pl.BlockSpec((S, E), lambda i: (0, 0)),
            pl.BlockSpec((S, E), lambda i: (0, 0)),
        ],
        out_specs=[
            pl.BlockSpec((S, 1), lambda i: (0, 0)),
            pl.BlockSpec((S, 1), lambda i: (0, 0)),
            pl.BlockSpec((1, E), lambda i: (0, 0)),
        ],
        out_shape=[
            jax.ShapeDtypeStruct((S, 1), jnp.int32),
            jax.ShapeDtypeStruct((S, 1), jnp.int32),
            jax.ShapeDtypeStruct((1, E), jnp.int32),
        ],
    )(oh1, oh2)

    pos0 = pos0c.reshape(S)
    pos1 = pos1c.reshape(S)
    start_pad = start_row.reshape(E)

    dispatch = functools.partial(
        pl.kernel,
        mesh=_sc_mesh(),
        out_type=jax.ShapeDtypeStruct((PT, H), jnp.float32),
        scratch_types=[
            pltpu.VMEM((TPW,), jnp.int32),
            pltpu.VMEM((TPW,), jnp.int32),
            pltpu.VMEM((TPW, H), jnp.float32),
            pltpu.SemaphoreType.DMA,
            pltpu.SemaphoreType.DMA,
        ],
    )(_dispatch_body)
    xs = dispatch(ln2_out, pos0, pos1)

    y = pl.pallas_call(
        _gmm_body,
        grid_spec=pltpu.PrefetchScalarGridSpec(
            num_scalar_prefetch=1,
            grid=(NB,),
            in_specs=[
                pl.BlockSpec((BLK, H), lambda b, s: (b, 0)),
                pl.BlockSpec((1, H, FFN), lambda b, s: (_expert_of(b, s), 0, 0)),
                pl.BlockSpec((1, FFN, H), lambda b, s: (_expert_of(b, s), 0, 0)),
            ],
            out_specs=pl.BlockSpec((BLK, H), lambda b, s: (b, 0)),
        ),
        out_shape=jax.ShapeDtypeStruct((PT, H), jnp.float32),
    )(start_pad, xs, moe_w1, moe_w2)

    combine = functools.partial(
        pl.kernel,
        mesh=_sc_mesh(),
        out_type=jax.ShapeDtypeStruct((S, H), jnp.float32),
        scratch_types=[
            pltpu.VMEM((_CHUNK,), jnp.int32),
            pltpu.VMEM((_CHUNK,), jnp.int32),
            pltpu.VMEM((_CHUNK, H), jnp.float32),
            pltpu.VMEM((_CHUNK, H), jnp.float32),
            pltpu.VMEM((_CHUNK, H), jnp.float32),
            pltpu.VMEM((_CHUNK, 16), jnp.float32),
            pltpu.VMEM((_CHUNK, 16), jnp.float32),
            pltpu.SemaphoreType.DMA,
            pltpu.SemaphoreType.DMA,
        ],
    )(_combine_body)
    out = combine(y, h2, pos0, pos1, g1c, g2c)

    return out.reshape(S, B, H)


# speedup vs baseline: 1.3327x; 1.0449x over previous
import functools

import jax
import jax.numpy as jnp
import numpy as np
from jax import lax
from jax.experimental import pallas as pl
from jax.experimental.pallas import tpu as pltpu
from jax.experimental.pallas import tpu_sc as plsc

S, B, H = 2048, 1, 1024
NH, NKV, DH = 16, 16, 64
E, TOPK, FFN = 16, 2, 1024

BLK = 128
PT = TOPK * S + E * BLK
NB = PT // BLK
NC, NS = 2, 16
NW = NC * NS
TPW = S // NW

_HIGH = jax.lax.Precision.DEFAULT


def _dot(a, b, dims, precision=_HIGH):
    return jax.lax.dot_general(a, b, (dims, ((), ())), precision=precision,
                               preferred_element_type=jnp.float32)


_TB1 = 256


def _ln_qkv_body(x_ref, w_ref, b_ref, qkvw_ref, o_ref):
    x = x_ref[...]
    mu = jnp.mean(x, axis=-1, keepdims=True)
    var = jnp.mean((x - mu) ** 2, axis=-1, keepdims=True)
    xn = (x - mu) / jnp.sqrt(var + 1e-5) * w_ref[...] + b_ref[...]
    o_ref[...] = _dot(xn, qkvw_ref[...], ((1,), (1,)))


_QB = 256


def _attn_body(q_ref, k_ref, v_ref, o_ref, *, scale):
    qb = q_ref[0]
    i = pl.program_id(1)

    def compute(kl):
        kb = k_ref[0, :kl, :]
        vb = v_ref[0, :kl, :]
        s = _dot(qb, kb, ((1,), (1,))) * scale
        row = i * _QB + jax.lax.broadcasted_iota(jnp.int32, (_QB, kl), 0)
        col = jax.lax.broadcasted_iota(jnp.int32, (_QB, kl), 1)
        s = jnp.where(col <= row, s, jnp.float32(-1e9))
        m = jnp.max(s, axis=-1, keepdims=True)
        p = jnp.exp(s - m)
        den = jnp.sum(p, axis=-1, keepdims=True)
        o_ref[0] = _dot(p, vb, ((1,), (0,))) / den

    for b in range(S // _QB):
        @pl.when(i == b)
        def _(b=b):
            compute((b + 1) * _QB)


_TB3 = 256


def _proj_gates_body(attn_ref, hid_ref, projw_ref, ln2w_ref, ln2b_ref,
                     routw_ref, h2_ref, ln2o_ref, oh1_ref, oh2_ref,
                     g1_ref, g2_ref):
    proj = _dot(attn_ref[...], projw_ref[...], ((1,), (1,)))
    h2 = hid_ref[...] + proj
    h2_ref[...] = h2
    mu = jnp.mean(h2, axis=-1, keepdims=True)
    var = jnp.mean((h2 - mu) ** 2, axis=-1, keepdims=True)
    ln2 = (h2 - mu) / jnp.sqrt(var + 1e-5) * ln2w_ref[...] + ln2b_ref[...]
    ln2o_ref[...] = ln2
    logits = _dot(ln2, routw_ref[...], ((1,), (1,)))
    m = jnp.max(logits, axis=-1, keepdims=True)
    ex = jnp.exp(logits - m)
    p = ex / jnp.sum(ex, axis=-1, keepdims=True)
    lanes = jax.lax.broadcasted_iota(jnp.int32, (_TB3, E), 1)
    i1 = jnp.argmax(p, axis=-1)[:, None]
    m1 = lanes == i1
    p_wo1 = jnp.where(m1, jnp.float32(-1.0), p)
    i2 = jnp.argmax(p_wo1, axis=-1)[:, None]
    m2 = lanes == i2
    oh1_ref[...] = m1.astype(jnp.float32)
    oh2_ref[...] = m2.astype(jnp.float32)
    g1 = jnp.sum(jnp.where(m1, p, 0.0), axis=-1, keepdims=True)
    g2 = jnp.sum(jnp.where(m2, p, 0.0), axis=-1, keepdims=True)
    g1_ref[...] = jnp.broadcast_to(g1, (_TB3, 16))
    g2_ref[...] = jnp.broadcast_to(g2, (_TB3, 16))


def _shift_down(x, d):
    return jnp.concatenate([jnp.zeros((d, E), jnp.float32), x[:S - d]], axis=0)


def _route_meta_body(oh1_ref, oh2_ref, pos0_ref, pos1_ref, start_ref):
    oh1 = oh1_ref[...]
    oh2 = oh2_ref[...]

    def incl_scan(x):
        acc = x
        d = 1
        while d < S:
            acc = acc + _shift_down(acc, d)
            d *= 2
        return acc

    inc1 = incl_scan(oh1)
    inc2 = incl_scan(oh2)
    within1 = inc1 - oh1
    within2 = inc2 - oh2
    counts0 = inc1[S - 1:S, :]
    counts = counts0 + inc2[S - 1:S, :]
    ci = counts.astype(jnp.int32)
    pc = ((ci + (BLK - 1)) // BLK) * BLK
    ltri = (jax.lax.broadcasted_iota(jnp.int32, (E, E), 0)
            < jax.lax.broadcasted_iota(jnp.int32, (E, E), 1)).astype(jnp.float32)
    start_f = _dot(pc.astype(jnp.float32), ltri, ((1,), (0,)))
    start_ref[...] = start_f.astype(jnp.int32)
    pos0 = jnp.sum(oh1 * (start_f + within1), axis=1, keepdims=True)
    pos1 = jnp.sum(oh2 * (start_f + counts0 + within2), axis=1, keepdims=True)
    pos0_ref[...] = pos0.astype(jnp.int32)
    pos1_ref[...] = pos1.astype(jnp.int32)


def _gmm_body(start_ref, x_ref, w1_ref, w2_ref, y_ref):
    h = _dot(x_ref[...], w1_ref[0], ((1,), (0,)))
    h = h * jax.nn.sigmoid(h)
    y_ref[...] = _dot(h, w2_ref[0], ((1,), (0,)))


def _expert_of(b, start_ref):
    acc = jnp.int32(0)
    for e in range(E):
        acc = acc + (start_ref[e] <= b * BLK).astype(jnp.int32)
    return acc - 1


def _sc_mesh():
    return plsc.VectorSubcoreMesh(core_axis_name="c", subcore_axis_name="s")


def _dispatch_body(ln2_hbm, pos0_hbm, pos1_hbm, xs_hbm,
                   idx0_v, idx1_v, rows_v, sem0, sem1):
    wid = lax.axis_index("s") * NC + lax.axis_index("c")
    base = wid * TPW
    pltpu.sync_copy(pos0_hbm.at[pl.ds(base, TPW)], idx0_v)
    pltpu.sync_copy(pos1_hbm.at[pl.ds(base, TPW)], idx1_v)
    pltpu.sync_copy(ln2_hbm.at[pl.ds(base, TPW)], rows_v)
    c0 = pltpu.async_copy(rows_v, xs_hbm.at[idx0_v], sem0)
    c1 = pltpu.async_copy(rows_v, xs_hbm.at[idx1_v], sem1)
    c0.wait()
    c1.wait()


_CHUNK = 32


def _combine_body(y_hbm, h2_hbm, pos0_hbm, pos1_hbm, g1_hbm, g2_hbm, out_hbm,
                  i0_v, i1_v, y0_v, y1_v, h2_v, g1_v, g2_v, sem0, sem1):
    wid = lax.axis_index("s") * NC + lax.axis_index("c")
    wbase = wid * TPW
    pltpu.sync_copy(pos0_hbm.at[pl.ds(wbase, TPW)], i0_v)
    pltpu.sync_copy(pos1_hbm.at[pl.ds(wbase, TPW)], i1_v)
    pltpu.sync_copy(g1_hbm.at[pl.ds(wbase, TPW)], g1_v)
    pltpu.sync_copy(g2_hbm.at[pl.ds(wbase, TPW)], g2_v)
    for c in range(TPW // _CHUNK):
        base = wbase + c * _CHUNK
        coff = c * _CHUNK
        c0 = pltpu.async_copy(y_hbm.at[i0_v.at[pl.ds(coff, _CHUNK)]], y0_v,
                              sem0)
        c1 = pltpu.async_copy(y_hbm.at[i1_v.at[pl.ds(coff, _CHUNK)]], y1_v,
                              sem1)
        pltpu.sync_copy(h2_hbm.at[pl.ds(base, _CHUNK)], h2_v)
        c0.wait()
        c1.wait()

        def body(i, carry):
            g1vec = g1_v[coff + i, :]
            g2vec = g2_v[coff + i, :]
            for j in range(H // 16):
                sl = pl.ds(j * 16, 16)
                h2_v[i, sl] = (h2_v[i, sl] + g1vec * y0_v[i, sl]
                               + g2vec * y1_v[i, sl])
            return carry

        lax.fori_loop(0, _CHUNK, body, 0)
        pltpu.sync_copy(h2_v, out_hbm.at[pl.ds(base, _CHUNK)])


def kernel(hidden_states, ln1_weight, ln1_bias, ln2_weight, ln2_bias,
           qkv_weight, proj_weight, router_weight, moe_w1, moe_w2):
    scale = 1.0 / np.sqrt(DH)
    x = hidden_states.reshape(S, H)

    qkv = pl.pallas_call(
        _ln_qkv_body,
        grid=(S // _TB1,),
        in_specs=[
            pl.BlockSpec((_TB1, H), lambda i: (i, 0)),
            pl.BlockSpec((1, H), lambda i: (0, 0)),
            pl.BlockSpec((1, H), lambda i: (0, 0)),
            pl.BlockSpec(((NH + 2 * NKV) * DH, H), lambda i: (0, 0)),
        ],
        out_specs=pl.BlockSpec((_TB1, (NH + 2 * NKV) * DH), lambda i: (i, 0)),
        out_shape=jax.ShapeDtypeStruct((S, (NH + 2 * NKV) * DH), jnp.float32),
    )(x, ln1_weight.reshape(1, H), ln1_bias.reshape(1, H), qkv_weight)

    qkvT = qkv.reshape(S, NH + 2 * NKV, DH).transpose(1, 0, 2)

    ctx = pl.pallas_call(
        functools.partial(_attn_body, scale=scale),
        grid=(NH, S // _QB),
        in_specs=[
            pl.BlockSpec((1, _QB, DH), lambda h, i: (h, i, 0)),
            pl.BlockSpec((1, S, DH), lambda h, i: (NH + h, 0, 0)),
            pl.BlockSpec((1, S, DH), lambda h, i: (NH + NKV + h, 0, 0)),
        ],
        out_specs=pl.BlockSpec((1, _QB, DH), lambda h, i: (h, i, 0)),
        out_shape=jax.ShapeDtypeStruct((NH, S, DH), jnp.float32),
    )(qkvT, qkvT, qkvT)

    attn_flat = ctx.transpose(1, 0, 2).reshape(S, NH * DH)

    h2, ln2_out, oh1, oh2, g1c, g2c = pl.pallas_call(
        _proj_gates_body,
        grid=(S // _TB3,),
        in_specs=[
            pl.BlockSpec((_TB3, NH * DH), lambda i: (i, 0)),
            pl.BlockSpec((_TB3, H), lambda i: (i, 0)),
            pl.BlockSpec((H, NH * DH), lambda i: (0, 0)),
            pl.BlockSpec((1, H), lambda i: (0, 0)),
            pl.BlockSpec((1, H), lambda i: (0, 0)),
            pl.BlockSpec((E, H), lambda i: (0, 0)),
        ],
        out_specs=[
            pl.BlockSpec((_TB3, H), lambda i: (i, 0)),
            pl.BlockSpec((_TB3, H), lambda i: (i, 0)),
            pl.BlockSpec((_TB3, E), lambda i: (i, 0)),
            pl.BlockSpec((_TB3, E), lambda i: (i, 0)),
            pl.BlockSpec((_TB3, 16), lambda i: (i, 0)),
            pl.BlockSpec((_TB3, 16), lambda i: (i, 0)),
        ],
        out_shape=[
            jax.ShapeDtypeStruct((S, H), jnp.float32),
            jax.ShapeDtypeStruct((S, H), jnp.float32),
            jax.ShapeDtypeStruct((S, E), jnp.float32),
            jax.ShapeDtypeStruct((S, E), jnp.float32),
            jax.ShapeDtypeStruct((S, 16), jnp.float32),
            jax.ShapeDtypeStruct((S, 16), jnp.float32),
        ],
    )(attn_flat, x, proj_weight, ln2_weight.reshape(1, H),
      ln2_bias.reshape(1, H), router_weight)

    pos0c, pos1c, start_row = pl.pallas_call(
        _route_meta_body,
        grid=(1,),
        in_specs=[
            pl.BlockSpec((S, E), lambda i: (0, 0)),
            pl.BlockSpec((S, E), lambda i: (0, 0)),
        ],
        out_specs=[
            pl.BlockSpec((S, 1), lambda i: (0, 0)),
            pl.BlockSpec((S, 1), lambda i: (0, 0)),
            pl.BlockSpec((1, E), lambda i: (0, 0)),
        ],
        out_shape=[
            jax.ShapeDtypeStruct((S, 1), jnp.int32),
            jax.ShapeDtypeStruct((S, 1), jnp.int32),
            jax.ShapeDtypeStruct((1, E), jnp.int32),
        ],
    )(oh1, oh2)

    pos0 = pos0c.reshape(S)
    pos1 = pos1c.reshape(S)
    start_pad = start_row.reshape(E)

    dispatch = functools.partial(
        pl.kernel,
        mesh=_sc_mesh(),
        out_type=jax.ShapeDtypeStruct((PT, H), jnp.float32),
        scratch_types=[
            pltpu.VMEM((TPW,), jnp.int32),
            pltpu.VMEM((TPW,), jnp.int32),
            pltpu.VMEM((TPW, H), jnp.float32),
            pltpu.SemaphoreType.DMA,
            pltpu.SemaphoreType.DMA,
        ],
    )(_dispatch_body)
    xs = dispatch(ln2_out, pos0, pos1)

    y = pl.pallas_call(
        _gmm_body,
        grid_spec=pltpu.PrefetchScalarGridSpec(
            num_scalar_prefetch=1,
            grid=(NB,),
            in_specs=[
                pl.BlockSpec((BLK, H), lambda b, s: (b, 0)),
                pl.BlockSpec((1, H, FFN), lambda b, s: (_expert_of(b, s), 0, 0)),
                pl.BlockSpec((1, FFN, H), lambda b, s: (_expert_of(b, s), 0, 0)),
            ],
            out_specs=pl.BlockSpec((BLK, H), lambda b, s: (b, 0)),
        ),
        out_shape=jax.ShapeDtypeStruct((PT, H), jnp.float32),
    )(start_pad, xs, moe_w1, moe_w2)

    combine = functools.partial(
        pl.kernel,
        mesh=_sc_mesh(),
        out_type=jax.ShapeDtypeStruct((S, H), jnp.float32),
        scratch_types=[
            pltpu.VMEM((TPW,), jnp.int32),
            pltpu.VMEM((TPW,), jnp.int32),
            pltpu.VMEM((_CHUNK, H), jnp.float32),
            pltpu.VMEM((_CHUNK, H), jnp.float32),
            pltpu.VMEM((_CHUNK, H), jnp.float32),
            pltpu.VMEM((TPW, 16), jnp.float32),
            pltpu.VMEM((TPW, 16), jnp.float32),
            pltpu.SemaphoreType.DMA,
            pltpu.SemaphoreType.DMA,
        ],
    )(_combine_body)
    out = combine(y, h2, pos0, pos1, g1c, g2c)

    return out.reshape(S, B, H)
